# Initial kernel scaffold; baseline (speedup 1.0000x reference)
import jax
import jax.numpy as jnp
from jax.experimental import pallas as pl


def kernel(node_input, edge_src, edge_dst, edge_attr, edge_scalar_attr, W_self, W_mlp1, W_mlp2, W_tp, W_out):
    raise NotImplementedError("write your pallas kernel here")


# speedup vs baseline: 2.2643x; 2.2643x over previous
import functools
import math

import jax
import jax.numpy as jnp
from jax import lax
from jax.experimental import pallas as pl
from jax.experimental.pallas import tpu as pltpu
from jax.experimental.pallas import tpu_sc as plsc

N = 10000
E = 320000
D = 128
DE = 16
H = 16
DOUT = 128
COS = math.cos(math.pi / 8)
SIN = math.sin(math.pi / 8)
INV_SQRT_NN = 1.0 / math.sqrt(32.0)
INV_SQRT_H = 1.0 / math.sqrt(float(H))

NC, NS = 2, 16
NW = NC * NS
EPW = E // NW
CHUNK = 80
NCHUNK = EPW // CHUNK
ZROWS = 80
NZB = N // ZROWS


def _self_body(x_ref, w_ref, nf_ref, nso_ref):
    t = jnp.dot(x_ref[...], w_ref[...], preferred_element_type=jnp.float32)
    nf_ref[...] = t[:, :D]
    nso_ref[...] = t[:, D:]


def _self_interaction(node_input, W_self):
    B = 1000
    return pl.pallas_call(
        _self_body,
        grid=(N // B,),
        in_specs=[
            pl.BlockSpec((B, D), lambda i: (i, 0)),
            pl.BlockSpec((D, D + DOUT), lambda i: (0, 0)),
        ],
        out_specs=[
            pl.BlockSpec((B, D), lambda i: (i, 0)),
            pl.BlockSpec((B, DOUT), lambda i: (i, 0)),
        ],
        out_shape=[
            jax.ShapeDtypeStruct((N, D), jnp.float32),
            jax.ShapeDtypeStruct((N, DOUT), jnp.float32),
        ],
    )(node_input, W_self)


def _mix_body(esa_ref, ea_ref, w1_ref, w2_ref, rep_ref, til_ref, wtp_ref,
              mix_ref):
    w = jax.nn.gelu(jnp.dot(esa_ref[...], w1_ref[...],
                            preferred_element_type=jnp.float32))
    w = jax.nn.gelu(jnp.dot(w, w2_ref[...],
                            preferred_element_type=jnp.float32))
    a = (jnp.dot(w, rep_ref[...], preferred_element_type=jnp.float32)
         * jnp.dot(ea_ref[...], til_ref[...],
                   preferred_element_type=jnp.float32))
    mix_ref[...] = jnp.dot(a, wtp_ref[...],
                           preferred_element_type=jnp.float32)


def _edge_mix(edge_scalar_attr, edge_attr, W_mlp1, W_mlp2, rep, til, W2s):
    B = 4000
    return pl.pallas_call(
        _mix_body,
        grid=(E // B,),
        in_specs=[
            pl.BlockSpec((B, DE), lambda i: (i, 0)),
            pl.BlockSpec((B, DE), lambda i: (i, 0)),
            pl.BlockSpec((DE, H), lambda i: (0, 0)),
            pl.BlockSpec((H, H), lambda i: (0, 0)),
            pl.BlockSpec((H, H * DE), lambda i: (0, 0)),
            pl.BlockSpec((DE, H * DE), lambda i: (0, 0)),
            pl.BlockSpec((H * DE, D), lambda i: (0, 0)),
        ],
        out_specs=pl.BlockSpec((B, D), lambda i: (i, 0)),
        out_shape=jax.ShapeDtypeStruct((E, D), jnp.float32),
    )(edge_scalar_attr, edge_attr, W_mlp1, W_mlp2, rep, til, W2s)


def _sc_body(nf_hbm, mix_hbm, src_hbm, dst_hbm, out_hbm,
             src_v, dst_v, rows_v, mix_v, agg_sh, sem_g, sem_m, sem_s, sem_d):
    c = lax.axis_index("c")
    s = lax.axis_index("s")
    wid = c * NS + s

    def zrow(r, carry):
        for g in range(D // 16):
            rows_v[r, pl.ds(16 * g, 16)] = jnp.zeros((16,), jnp.float32)
        return carry

    lax.fori_loop(0, ZROWS, zrow, 0)
    for k in range((NZB + NS - 1) // NS):
        b = s + NS * k
        @pl.when(b < NZB)
        def _():
            pltpu.sync_copy(rows_v, agg_sh.at[pl.ds(b * ZROWS, ZROWS)])
    plsc.subcore_barrier()

    def chunk(j, carry):
        eoff = wid * EPW + j * CHUNK
        cp_s = pltpu.async_copy(src_hbm.at[wid, j], src_v, sem_s)
        cp_d = pltpu.async_copy(dst_hbm.at[wid, j], dst_v, sem_d)
        cp_m = pltpu.async_copy(mix_hbm.at[pl.ds(eoff, CHUNK)], mix_v, sem_m)
        cp_s.wait()
        cp_g = pltpu.async_copy(nf_hbm.at[src_v.at[0]], rows_v, sem_g)
        cp_g.wait()
        cp_m.wait()

        def mrow(r, inner):
            for g in range(D // 16):
                sl = pl.ds(16 * g, 16)
                rows_v[r, sl] = rows_v[r, sl] * mix_v[r, sl]
            return inner

        lax.fori_loop(0, CHUNK, mrow, 0)
        cp_d.wait()
        pltpu.sync_copy(rows_v, agg_sh.at[dst_v.at[0]], add=True)
        return carry

    lax.fori_loop(0, NCHUNK, chunk, 0)
    plsc.subcore_barrier()

    for k in range((NZB + NS - 1) // NS):
        b = s + NS * k
        @pl.when(b < NZB)
        def _():
            pltpu.sync_copy(agg_sh.at[pl.ds(b * ZROWS, ZROWS)], rows_v)
            pltpu.sync_copy(rows_v, out_hbm.at[c, pl.ds(b * ZROWS, ZROWS)])


_sc_scatter = functools.partial(
    pl.kernel,
    out_type=jax.ShapeDtypeStruct((NC, N, D), jnp.float32),
    mesh=plsc.VectorSubcoreMesh(core_axis_name="c", subcore_axis_name="s"),
    scratch_types=[
        pltpu.VMEM((1, CHUNK), jnp.int32),
        pltpu.VMEM((1, CHUNK), jnp.int32),
        pltpu.VMEM((CHUNK, D), jnp.float32),
        pltpu.VMEM((CHUNK, D), jnp.float32),
        pltpu.VMEM_SHARED((N, D), jnp.float32),
        pltpu.SemaphoreType.DMA,
        pltpu.SemaphoreType.DMA,
        pltpu.SemaphoreType.DMA,
        pltpu.SemaphoreType.DMA,
    ],
)(_sc_body)


def _post_body(nso_ref, a0_ref, a1_ref, w_ref, o_ref):
    agg = a0_ref[...] + a1_ref[...]
    o_ref[...] = (COS * nso_ref[...]
                  + jnp.dot(agg, w_ref[...],
                            preferred_element_type=jnp.float32))


def _post(nso, agg0, agg1, W_out_scaled):
    B = 1000
    return pl.pallas_call(
        _post_body,
        grid=(N // B,),
        in_specs=[
            pl.BlockSpec((B, DOUT), lambda i: (i, 0)),
            pl.BlockSpec((B, D), lambda i: (i, 0)),
            pl.BlockSpec((B, D), lambda i: (i, 0)),
            pl.BlockSpec((D, DOUT), lambda i: (0, 0)),
        ],
        out_specs=pl.BlockSpec((B, DOUT), lambda i: (i, 0)),
        out_shape=jax.ShapeDtypeStruct((N, DOUT), jnp.float32),
    )(nso, agg0, agg1, W_out_scaled)


def kernel(node_input, edge_src, edge_dst, edge_attr, edge_scalar_attr,
           W_self, W_mlp1, W_mlp2, W_tp, W_out):
    nf, nso = _self_interaction(node_input, W_self)

    eye = jnp.eye(H, dtype=jnp.float32)
    rep = jnp.repeat(eye, DE, axis=1)
    til = jnp.tile(jnp.eye(DE, dtype=jnp.float32), (1, H))
    W2s = (jnp.transpose(W_tp, (0, 2, 1)).reshape(H * DE, D) * INV_SQRT_H)

    mix = _edge_mix(edge_scalar_attr, edge_attr, W_mlp1, W_mlp2, rep, til,
                    W2s)

    src4d = edge_src.reshape(NW, NCHUNK, 1, CHUNK)
    dst4d = edge_dst.reshape(NW, NCHUNK, 1, CHUNK)
    aggs = _sc_scatter(nf, mix, src4d, dst4d)

    return _post(nso, aggs[0], aggs[1], W_out * (SIN * INV_SQRT_NN))
